# Initial kernel scaffold; baseline (speedup 1.0000x reference)
import jax
import jax.numpy as jnp
from jax.experimental import pallas as pl


def kernel(h, edge_index, W1, b1, W2, b2, Wo, bo):
    raise NotImplementedError("write your pallas kernel here")


# speedup vs baseline: 35.1170x; 35.1170x over previous
import functools

import jax
import jax.numpy as jnp
from jax import lax
from jax.experimental import pallas as pl
from jax.experimental.pallas import tpu as pltpu
from jax.experimental.pallas import tpu_sc as plsc

N = 10000
H = 16
E = 320000
NC, NS = 2, 16
NW = NC * NS
CH = 128
NCHUNK = 20
E_WP = NCHUNK * CH
E_PAD = NW * E_WP

NCHUNK = 80
E_WP = NCHUNK * CH
E_PAD = NW * E_WP
NPAD = 10240
RPS = NPAD // NS

_mesh = plsc.VectorSubcoreMesh(core_axis_name="c", subcore_axis_name="s")


@functools.partial(
    pl.kernel,
    mesh=_mesh,
    out_type=jax.ShapeDtypeStruct((NC, NPAD), jnp.float32),
    scratch_types=[
        pltpu.VMEM((NCHUNK, CH), jnp.int32),
        pltpu.VMEM((CH,), jnp.float32),
        pltpu.VMEM_SHARED((NPAD,), jnp.float32),
    ],
)
def _deg_kernel(dst_hbm, zeros_hbm, out_hbm, dst_v, ones_v, acc_sh):
    cid = lax.axis_index("c")
    sid = lax.axis_index("s")
    wid = sid * NC + cid
    pltpu.sync_copy(zeros_hbm.at[pl.ds(sid * RPS, RPS)],
                    acc_sh.at[pl.ds(sid * RPS, RPS)])
    for i in range(CH // 16):
        ones_v[pl.ds(i * 16, 16)] = jnp.ones((16,), jnp.float32)
    pltpu.sync_copy(dst_hbm.at[wid], dst_v)
    plsc.subcore_barrier()

    def body(j, carry):
        pltpu.sync_copy(ones_v, acc_sh.at[dst_v.at[j]], add=True)
        return carry

    lax.fori_loop(0, NCHUNK, body, 0)
    plsc.subcore_barrier()
    pltpu.sync_copy(acc_sh.at[pl.ds(sid * RPS, RPS)],
                    out_hbm.at[cid, pl.ds(sid * RPS, RPS)])


@functools.partial(
    pl.kernel,
    mesh=_mesh,
    compiler_params=pltpu.CompilerParams(use_tc_tiling_on_sc=False),
    out_type=jax.ShapeDtypeStruct((NC, NPAD, H), jnp.float32),
    scratch_types=[
        pltpu.VMEM((NCHUNK, CH), jnp.int32),
        pltpu.VMEM((NCHUNK, CH), jnp.int32),
        pltpu.VMEM((CH, H), jnp.float32),
        pltpu.VMEM_SHARED((NPAD, H), jnp.float32),
        pltpu.SemaphoreType.DMA,
    ],
)
def _sweep_kernel(y_hbm, src_hbm, dst_hbm, zeros_hbm, out_hbm,
                  src_v, dst_v, rows_v, acc_sh, sem):
    cid = lax.axis_index("c")
    sid = lax.axis_index("s")
    wid = sid * NC + cid
    pltpu.sync_copy(zeros_hbm.at[pl.ds(sid * RPS, RPS)],
                    acc_sh.at[pl.ds(sid * RPS, RPS)])
    pltpu.sync_copy(src_hbm.at[wid], src_v)
    pltpu.sync_copy(dst_hbm.at[wid], dst_v)
    plsc.subcore_barrier()

    def body(j, carry):
        pltpu.async_copy(y_hbm.at[src_v.at[j]], rows_v, sem).wait()
        pltpu.sync_copy(rows_v, acc_sh.at[dst_v.at[j]], add=True)
        return carry

    lax.fori_loop(0, NCHUNK, body, 0)
    plsc.subcore_barrier()
    pltpu.sync_copy(acc_sh.at[pl.ds(sid * RPS, RPS)],
                    out_hbm.at[cid, pl.ds(sid * RPS, RPS)])


def _tc1_body(h_ref, w1_ref, degp_ref, y1_ref, xw1_ref, dv_ref):
    deg = degp_ref[0] + degp_ref[1] + 1.0
    dinv = lax.rsqrt(deg)[:N]
    xw = jnp.dot(h_ref[...], w1_ref[...], preferred_element_type=jnp.float32)
    xw1_ref[...] = xw
    y1_ref[...] = xw * dinv
    dv_ref[...] = dinv


def _tc2_body(accp_ref, xw1_ref, dv_ref, b1_ref, w2_ref, y2_ref, xw2_ref):
    s = accp_ref[0, :N] + accp_ref[1, :N]
    dv = dv_ref[...]
    x1 = jnp.maximum(dv * s + (dv * dv) * xw1_ref[...] + b1_ref[...][None, :],
                     0.0)
    xw2 = jnp.dot(x1, w2_ref[...], preferred_element_type=jnp.float32)
    xw2_ref[...] = xw2
    y2_ref[...] = xw2 * dv


def _tc3_body(accp_ref, xw2_ref, dv_ref, b2_ref, wo_ref, bo_ref, out_ref):
    s = accp_ref[0, :N] + accp_ref[1, :N]
    dv = dv_ref[...]
    x2 = jnp.maximum(dv * s + (dv * dv) * xw2_ref[...] + b2_ref[...][None, :],
                     0.0)
    out_ref[...] = (jnp.dot(x2, wo_ref[...], preferred_element_type=jnp.float32)
                    + bo_ref[...][None, :])


_tc1 = pl.pallas_call(
    _tc1_body,
    out_shape=(
        jax.ShapeDtypeStruct((N, H), jnp.float32),
        jax.ShapeDtypeStruct((N, H), jnp.float32),
        jax.ShapeDtypeStruct((N, 1), jnp.float32),
    ),
)

_tc2 = pl.pallas_call(
    _tc2_body,
    out_shape=(
        jax.ShapeDtypeStruct((N, H), jnp.float32),
        jax.ShapeDtypeStruct((N, H), jnp.float32),
    ),
)

_tc3 = pl.pallas_call(
    _tc3_body,
    out_shape=jax.ShapeDtypeStruct((N, 1), jnp.float32),
)


@jax.jit
def kernel(h, edge_index, W1, b1, W2, b2, Wo, bo):
    src = edge_index[0].astype(jnp.int32)
    dst = edge_index[1].astype(jnp.int32)
    npad = E_PAD - E
    pad = jnp.arange(npad, dtype=jnp.int32)
    src_p = jnp.concatenate([src, pad % N]).reshape(NW, NCHUNK, CH)
    dst_p = jnp.concatenate([dst, N + pad % (NPAD - N)]).reshape(NW, NCHUNK, CH)

    zeros1 = jnp.zeros((NPAD,), jnp.float32)
    zeros2 = jnp.zeros((NPAD, H), jnp.float32)

    degp = _deg_kernel(dst_p, zeros1)
    y1, xw1, dv = _tc1(h, W1, degp.reshape(NC, NPAD, 1))
    acc1 = _sweep_kernel(y1, src_p, dst_p, zeros2)
    y2, xw2 = _tc2(acc1, xw1, dv, b1, W2)
    acc2 = _sweep_kernel(y2, src_p, dst_p, zeros2)
    return _tc3(acc2, xw2, dv, b2, Wo, bo)


# speedup vs baseline: 59.2380x; 1.6869x over previous
import functools

import jax
import jax.numpy as jnp
from jax import lax
from jax.experimental import pallas as pl
from jax.experimental.pallas import tpu as pltpu
from jax.experimental.pallas import tpu_sc as plsc

N = 10000
H = 16
E = 320000
NC, NS = 2, 16
NW = NC * NS
CH = 128
NCHUNK = 20
E_WP = NCHUNK * CH
E_PAD = NW * E_WP

NCHUNK = 80
E_WP = NCHUNK * CH
E_PAD = NW * E_WP
NPAD = 10240
RPS = NPAD // NS

_mesh = plsc.VectorSubcoreMesh(core_axis_name="c", subcore_axis_name="s")


@functools.partial(
    pl.kernel,
    mesh=_mesh,
    out_type=jax.ShapeDtypeStruct((NC, NPAD), jnp.float32),
    scratch_types=[
        pltpu.VMEM((E_WP,), jnp.int32),
        pltpu.VMEM((E_WP,), jnp.float32),
        pltpu.VMEM_SHARED((NPAD,), jnp.float32),
    ],
)
def _deg_kernel(dst_hbm, zeros_hbm, ones_hbm, out_hbm, dst_v, ones_v, acc_sh):
    cid = lax.axis_index("c")
    sid = lax.axis_index("s")
    wid = sid * NC + cid
    pltpu.sync_copy(zeros_hbm.at[pl.ds(sid * RPS, RPS)],
                    acc_sh.at[pl.ds(sid * RPS, RPS)])
    pltpu.sync_copy(ones_hbm, ones_v)
    pltpu.sync_copy(dst_hbm.at[wid], dst_v)
    plsc.subcore_barrier()
    pltpu.sync_copy(ones_v, acc_sh.at[dst_v], add=True)
    plsc.subcore_barrier()
    pltpu.sync_copy(acc_sh.at[pl.ds(sid * RPS, RPS)],
                    out_hbm.at[cid, pl.ds(sid * RPS, RPS)])


KG = 8
NB = NCHUNK // KG
KGCH = KG * CH


@functools.partial(
    pl.kernel,
    mesh=_mesh,
    compiler_params=pltpu.CompilerParams(use_tc_tiling_on_sc=False),
    out_type=jax.ShapeDtypeStruct((NC, NPAD, H), jnp.float32),
    scratch_types=[
        pltpu.VMEM((NB, KGCH), jnp.int32),
        pltpu.VMEM((NB, KGCH), jnp.int32),
        pltpu.VMEM((2, KGCH, H), jnp.float32),
        pltpu.VMEM_SHARED((NPAD, H), jnp.float32),
        pltpu.SemaphoreType.DMA,
        pltpu.SemaphoreType.DMA,
    ],
)
def _sweep_kernel(y_hbm, src_hbm, dst_hbm, zeros_hbm, out_hbm,
                  src_v, dst_v, rows_v, acc_sh, semA, semB):
    cid = lax.axis_index("c")
    sid = lax.axis_index("s")
    wid = sid * NC + cid
    pltpu.sync_copy(zeros_hbm.at[pl.ds(sid * RPS, RPS)],
                    acc_sh.at[pl.ds(sid * RPS, RPS)])
    pltpu.sync_copy(src_hbm.at[wid], src_v)
    pltpu.sync_copy(dst_hbm.at[wid], dst_v)
    pltpu.async_copy(y_hbm.at[src_v.at[0]], rows_v.at[0], semA)
    plsc.subcore_barrier()

    def body(i, carry):
        j0 = 2 * i
        j1 = j0 + 1
        pltpu.async_copy(y_hbm.at[src_v.at[j1]], rows_v.at[1], semB)
        pltpu.make_async_copy(y_hbm.at[src_v.at[j0]], rows_v.at[0], semA).wait()
        pltpu.sync_copy(rows_v.at[0], acc_sh.at[dst_v.at[j0]], add=True)

        @pl.when(j0 + 2 < NB)
        def _():
            pltpu.async_copy(y_hbm.at[src_v.at[j0 + 2]], rows_v.at[0], semA)

        pltpu.make_async_copy(y_hbm.at[src_v.at[j1]], rows_v.at[1], semB).wait()
        pltpu.sync_copy(rows_v.at[1], acc_sh.at[dst_v.at[j1]], add=True)
        return carry

    lax.fori_loop(0, NB // 2, body, 0)
    plsc.subcore_barrier()
    pltpu.sync_copy(acc_sh.at[pl.ds(sid * RPS, RPS)],
                    out_hbm.at[cid, pl.ds(sid * RPS, RPS)])


def _tc1_body(h_ref, w1_ref, degp_ref, y1_ref, xw1_ref, dv_ref):
    deg = degp_ref[0] + degp_ref[1] + 1.0
    dinv = lax.rsqrt(deg)[:N]
    xw = jnp.dot(h_ref[...], w1_ref[...], preferred_element_type=jnp.float32)
    xw1_ref[...] = xw
    y1_ref[...] = xw * dinv
    dv_ref[...] = dinv


def _tc2_body(accp_ref, xw1_ref, dv_ref, b1_ref, w2_ref, y2_ref, xw2_ref):
    s = accp_ref[0, :N] + accp_ref[1, :N]
    dv = dv_ref[...]
    x1 = jnp.maximum(dv * s + (dv * dv) * xw1_ref[...] + b1_ref[...][None, :],
                     0.0)
    xw2 = jnp.dot(x1, w2_ref[...], preferred_element_type=jnp.float32)
    xw2_ref[...] = xw2
    y2_ref[...] = xw2 * dv


def _tc3_body(accp_ref, xw2_ref, dv_ref, b2_ref, wo_ref, bo_ref, out_ref):
    s = accp_ref[0, :N] + accp_ref[1, :N]
    dv = dv_ref[...]
    x2 = jnp.maximum(dv * s + (dv * dv) * xw2_ref[...] + b2_ref[...][None, :],
                     0.0)
    out_ref[...] = (jnp.dot(x2, wo_ref[...], preferred_element_type=jnp.float32)
                    + bo_ref[...][None, :])


_tc1 = pl.pallas_call(
    _tc1_body,
    out_shape=(
        jax.ShapeDtypeStruct((N, H), jnp.float32),
        jax.ShapeDtypeStruct((N, H), jnp.float32),
        jax.ShapeDtypeStruct((N, 1), jnp.float32),
    ),
)

_tc2 = pl.pallas_call(
    _tc2_body,
    out_shape=(
        jax.ShapeDtypeStruct((N, H), jnp.float32),
        jax.ShapeDtypeStruct((N, H), jnp.float32),
    ),
)

_tc3 = pl.pallas_call(
    _tc3_body,
    out_shape=jax.ShapeDtypeStruct((N, 1), jnp.float32),
)


@jax.jit
def kernel(h, edge_index, W1, b1, W2, b2, Wo, bo):
    src = edge_index[0].astype(jnp.int32)
    dst = edge_index[1].astype(jnp.int32)
    npad = E_PAD - E
    pad = jnp.arange(npad, dtype=jnp.int32)
    src_p = jnp.concatenate([src, pad % N]).reshape(NW, NB, KGCH)
    dst_p = jnp.concatenate([dst, N + pad % (NPAD - N)]).reshape(NW, NB, KGCH)

    zeros1 = jnp.zeros((NPAD,), jnp.float32)
    zeros2 = jnp.zeros((NPAD, H), jnp.float32)
    ones2 = jnp.ones((E_WP,), jnp.float32)

    degp = _deg_kernel(dst_p.reshape(NW, E_WP), zeros1, ones2)
    y1, xw1, dv = _tc1(h, W1, degp.reshape(NC, NPAD, 1))
    acc1 = _sweep_kernel(y1, src_p, dst_p, zeros2)
    y2, xw2 = _tc2(acc1, xw1, dv, b1, W2)
    acc2 = _sweep_kernel(y2, src_p, dst_p, zeros2)
    return _tc3(acc2, xw2, dv, b2, Wo, bo)
